# Initial kernel scaffold; baseline (speedup 1.0000x reference)
import jax
import jax.numpy as jnp
from jax.experimental import pallas as pl


def kernel(feat_h, feat_p, feat_hp, eidx_h, eidx_p, eidx_hp, emb_h, emb_p, emb_hp, Ws1, Wn1, b1, a1, g1, be1, Ws, Wn, b, a, g, be):
    raise NotImplementedError("write your pallas kernel here")


# speedup vs baseline: 2.0475x; 2.0475x over previous
import functools

import jax
import jax.numpy as jnp
from jax import lax
from jax.experimental import pallas as pl
from jax.experimental.pallas import tpu as pltpu
from jax.experimental.pallas import tpu_sc as plsc

N = 10000
E = 160000
EMB = 64
H = 128
VOCAB = 257

NC = 2
NS = 16
NW = NC * NS
CH = 128
NCHUNK = -(-E // (NW * CH))
EPW = NCHUNK * CH
E_PAD = EPW * NW
NPAD = 10240
RPT = NPAD // NS
LC = 80
NLC = 4
LPW = NLC * LC
N_PADL = LPW * NW
CW = 8
BLK = 1000


def _mesh():
  return plsc.VectorSubcoreMesh(core_axis_name="c", subcore_axis_name="s")


_SC_PARAMS = pltpu.CompilerParams(use_tc_tiling_on_sc=False)


def _emb_lookup(e0, e1, e2, f_r):
  def body(e0r, e1r, e2r, fr, out, idxb, rowsb, sem):
    c = lax.axis_index("c")
    s = lax.axis_index("s")
    wid = s * NC + c
    base = wid * LPW
    for t, et in enumerate((e0r, e1r, e2r)):
      pltpu.sync_copy(fr.at[t, wid], idxb)
      for j in range(NLC):
        pltpu.async_copy(et.at[idxb.at[j]], rowsb, sem).wait()
        pltpu.sync_copy(rowsb, out.at[t, pl.ds(base + j * LC, LC)])

  kern = pl.kernel(
      body,
      out_type=jax.ShapeDtypeStruct((3, N_PADL, EMB), jnp.float32),
      mesh=_mesh(),
      compiler_params=_SC_PARAMS,
      scratch_types=[
          pltpu.VMEM((NLC, LC), jnp.int32),
          pltpu.VMEM((LC, EMB), jnp.float32),
          pltpu.SemaphoreType.DMA,
      ],
  )
  return kern(e0, e1, e2, f_r)


def _seg_sum(D, do_cnt, x, src_r, dst_r, zrows, zcnt=None, ones=None):
  def body(*refs):
    if do_cnt:
      (x_r, src_h, dst_h, zr, zc, on, out_s, out_c,
       acc, srcb, dstb, rows, sem, accc, onesv) = refs
    else:
      (x_r, src_h, dst_h, zr,
       out_s, acc, srcb, dstb, rows, sem) = refs
    c = lax.axis_index("c")
    s = lax.axis_index("s")
    wid = s * NC + c
    r0 = s * RPT
    pltpu.sync_copy(zr, acc.at[pl.ds(r0, RPT)])
    pltpu.sync_copy(src_h.at[wid], srcb)
    pltpu.sync_copy(dst_h.at[wid], dstb)
    if do_cnt:
      pltpu.sync_copy(zc, accc.at[pl.ds(r0, RPT)])
      pltpu.sync_copy(on, onesv)
    plsc.subcore_barrier()

    def step(j, carry):
      pltpu.async_copy(x_r.at[srcb.at[j]], rows, sem).wait()
      pltpu.sync_copy(rows, acc.at[dstb.at[j]], add=True)
      if do_cnt:
        pltpu.sync_copy(onesv, accc.at[dstb.at[j]], add=True)
      return carry

    lax.fori_loop(0, NCHUNK, step, 0)
    plsc.subcore_barrier()
    pltpu.sync_copy(acc.at[pl.ds(r0, RPT)], out_s.at[c, pl.ds(r0, RPT)])
    if do_cnt:
      pltpu.sync_copy(accc.at[pl.ds(r0, RPT)], out_c.at[c, pl.ds(r0, RPT)])

  out_type = [jax.ShapeDtypeStruct((NC, NPAD, D), jnp.float32)]
  scratch = [
      pltpu.VMEM_SHARED((NPAD, D), jnp.float32),
      pltpu.VMEM((NCHUNK, CH), jnp.int32),
      pltpu.VMEM((NCHUNK, CH), jnp.int32),
      pltpu.VMEM((CH, D), jnp.float32),
      pltpu.SemaphoreType.DMA,
  ]
  args = [x, src_r, dst_r, zrows]
  if do_cnt:
    out_type.append(jax.ShapeDtypeStruct((NC, NPAD, CW), jnp.float32))
    scratch += [pltpu.VMEM_SHARED((NPAD, CW), jnp.float32),
                pltpu.VMEM((CH, CW), jnp.float32)]
    args += [zcnt, ones]

  kern = pl.kernel(body, out_type=tuple(out_type), mesh=_mesh(),
                   compiler_params=_SC_PARAMS, scratch_types=scratch)
  return kern(*args)


def _tc_layer(x, s2, c2, ws, wn, pp):
  Din = x.shape[1]

  def body(xr, sr, cr, wsr, wnr, ppr, hr, gr):
    sv = sr[...]
    cv = cr[...]
    cnt = cv[0][:, 0:1] + cv[1][:, 0:1]
    hn = (sv[0] + sv[1]) / jnp.maximum(cnt, 1.0)
    r = jnp.dot(xr[...], wsr[...], preferred_element_type=jnp.float32)
    r = r + jnp.dot(hn, wnr[...], preferred_element_type=jnp.float32)
    r = r + ppr[0:1, :]
    r = jnp.where(r > 0, r, ppr[1:2, :] * r)
    r = ppr[2:3, :] * r + ppr[3:4, :]
    hr[...] = r

    @pl.when(pl.program_id(0) == 0)
    def _():
      gr[...] = jnp.zeros_like(gr)

    gr[...] += jnp.sum(r, axis=0, keepdims=True) * (1.0 / N)

  return pl.pallas_call(
      body,
      grid=(N // BLK,),
      in_specs=[
          pl.BlockSpec((BLK, Din), lambda i: (i, 0)),
          pl.BlockSpec((NC, BLK, Din), lambda i: (0, i, 0)),
          pl.BlockSpec((NC, BLK, CW), lambda i: (0, i, 0)),
          pl.BlockSpec((Din, H), lambda i: (0, 0)),
          pl.BlockSpec((Din, H), lambda i: (0, 0)),
          pl.BlockSpec((8, H), lambda i: (0, 0)),
      ],
      out_specs=[
          pl.BlockSpec((BLK, H), lambda i: (i, 0)),
          pl.BlockSpec((1, H), lambda i: (0, 0)),
      ],
      out_shape=[
          jax.ShapeDtypeStruct((N, H), jnp.float32),
          jax.ShapeDtypeStruct((1, H), jnp.float32),
      ],
  )(x, s2, c2, ws, wn, pp)


def kernel(feat_h, feat_p, feat_hp, eidx_h, eidx_p, eidx_hp,
           emb_h, emb_p, emb_hp, Ws1, Wn1, b1, a1, g1, be1,
           Ws, Wn, b, a, g, be):
  f = jnp.stack([feat_h, feat_p, feat_hp]).astype(jnp.int32)
  f = jnp.pad(f, ((0, 0), (0, N_PADL - N)))
  f_r = f.reshape(3, NW, NLC, LC)

  srcs, dsts = [], []
  for eidx in (eidx_h, eidx_p, eidx_hp):
    ei = eidx.astype(jnp.int32)
    srcp = jnp.pad(ei[0], (0, E_PAD - E))
    dstp = jnp.pad(ei[1], (0, E_PAD - E), constant_values=N)
    srcs.append(srcp.reshape(NW, NCHUNK, CH))
    dsts.append(dstp.reshape(NW, NCHUNK, CH))

  zrows_h = jnp.zeros((RPT, H), jnp.float32)
  zrows_e = jnp.zeros((RPT, EMB), jnp.float32)
  zcnt = jnp.zeros((RPT, CW), jnp.float32)
  ones = jnp.ones((CH, CW), jnp.float32)

  x0 = _emb_lookup(emb_h, emb_p, emb_hp, f_r)

  def pack_params(bb, aa, gg, bee):
    return jnp.concatenate(
        [jnp.stack([bb, aa, gg, bee]), jnp.zeros((4, H), jnp.float32)], axis=0)

  gsums = []
  for t in range(3):
    xt = x0[t]
    s2, c2 = _seg_sum(EMB, True, xt, srcs[t], dsts[t], zrows_e, zcnt, ones)
    h, gs = _tc_layer(xt, s2, c2, Ws1[t], Wn1[t],
                      pack_params(b1[t], a1[t], g1[t], be1[t]))
    t_gs = [gs]
    for l in range(3):
      (s2,) = _seg_sum(H, False, h, srcs[t], dsts[t], zrows_h)
      h, gs = _tc_layer(h, s2, c2, Ws[l, t], Wn[l, t],
                        pack_params(b[l, t], a[l, t], g[l, t], be[l, t]))
      t_gs.append(gs)
    gsums.append(t_gs)

  g_vec = jnp.concatenate(
      [gsums[t][l].reshape(H) for t in range(3) for l in range(4)])
  return g_vec.reshape(1, 12 * H)


# speedup vs baseline: 2.3316x; 1.1388x over previous
import functools

import jax
import jax.numpy as jnp
from jax import lax
from jax.experimental import pallas as pl
from jax.experimental.pallas import tpu as pltpu
from jax.experimental.pallas import tpu_sc as plsc

N = 10000
E = 160000
EMB = 64
H = 128
VOCAB = 257

NC = 2
NS = 16
NW = NC * NS
CH = 128
NCHUNK = -(-E // (NW * CH))
EPW = NCHUNK * CH
E_PAD = EPW * NW
NPAD = 10240
RPT = NPAD // NS
LC = 80
NLC = 4
LPW = NLC * LC
N_PADL = LPW * NW
CW = 8
BLK = 1000


def _mesh():
  return plsc.VectorSubcoreMesh(core_axis_name="c", subcore_axis_name="s")


_SC_PARAMS = pltpu.CompilerParams(use_tc_tiling_on_sc=False)


def _emb_lookup(e0, e1, e2, f_r):
  def body(e0r, e1r, e2r, fr, out, idxb, rowsb, sem):
    c = lax.axis_index("c")
    s = lax.axis_index("s")
    wid = s * NC + c
    base = wid * LPW
    for t, et in enumerate((e0r, e1r, e2r)):
      pltpu.sync_copy(fr.at[t, wid], idxb)
      for j in range(NLC):
        pltpu.async_copy(et.at[idxb.at[j]], rowsb, sem).wait()
        pltpu.sync_copy(rowsb, out.at[t, pl.ds(base + j * LC, LC)])

  kern = pl.kernel(
      body,
      out_type=jax.ShapeDtypeStruct((3, N_PADL, EMB), jnp.float32),
      mesh=_mesh(),
      compiler_params=_SC_PARAMS,
      scratch_types=[
          pltpu.VMEM((NLC, LC), jnp.int32),
          pltpu.VMEM((LC, EMB), jnp.float32),
          pltpu.SemaphoreType.DMA,
      ],
  )
  return kern(e0, e1, e2, f_r)


def _seg_sum(D, do_cnt, x, src_r, dst_r, zcnt=None, ones=None):
  def body(*refs):
    if do_cnt:
      (x_r, src_h, dst_h, zc, on, out_s, out_c,
       acc, srcb, dstb, rows, sem, accc, onesv) = refs
    else:
      (x_r, src_h, dst_h,
       out_s, acc, srcb, dstb, rows, sem) = refs
    c = lax.axis_index("c")
    s = lax.axis_index("s")
    wid = s * NC + c
    r0 = s * RPT

    zero16 = jnp.zeros((16,), jnp.float32)

    def zfill(i, carry):
      for k2 in range(D // 16):
        rows[0, i, pl.ds(k2 * 16, 16)] = zero16
      return carry

    lax.fori_loop(0, CH, zfill, 0)

    def zcp(i, carry):
      pltpu.sync_copy(rows.at[0], acc.at[pl.ds(r0 + i * CH, CH)])
      return carry

    lax.fori_loop(0, RPT // CH, zcp, 0)
    pltpu.sync_copy(src_h.at[wid], srcb)
    pltpu.sync_copy(dst_h.at[wid], dstb)
    if do_cnt:
      pltpu.sync_copy(zc, accc.at[pl.ds(r0, RPT)])
      pltpu.sync_copy(on, onesv)
    plsc.subcore_barrier()

    pltpu.async_copy(x_r.at[srcb.at[0]], rows.at[0], sem)

    def step(j, carry):
      nxt = j + 1

      @pl.when(nxt < NCHUNK)
      def _():
        pltpu.async_copy(x_r.at[srcb.at[nxt]], rows.at[nxt % 2], sem)

      pltpu.make_async_copy(x_r.at[srcb.at[j]], rows.at[j % 2], sem).wait()
      pltpu.sync_copy(rows.at[j % 2], acc.at[dstb.at[j]], add=True)
      if do_cnt:
        pltpu.sync_copy(onesv, accc.at[dstb.at[j]], add=True)
      return carry

    lax.fori_loop(0, NCHUNK, step, 0)
    plsc.subcore_barrier()
    pltpu.sync_copy(acc.at[pl.ds(r0, RPT)], out_s.at[c, pl.ds(r0, RPT)])
    if do_cnt:
      pltpu.sync_copy(accc.at[pl.ds(r0, RPT)], out_c.at[c, pl.ds(r0, RPT)])

  out_type = [jax.ShapeDtypeStruct((NC, NPAD, D), jnp.float32)]
  scratch = [
      pltpu.VMEM_SHARED((NPAD, D), jnp.float32),
      pltpu.VMEM((NCHUNK, CH), jnp.int32),
      pltpu.VMEM((NCHUNK, CH), jnp.int32),
      pltpu.VMEM((2, CH, D), jnp.float32),
      pltpu.SemaphoreType.DMA,
  ]
  args = [x, src_r, dst_r]
  if do_cnt:
    out_type.append(jax.ShapeDtypeStruct((NC, NPAD, CW), jnp.float32))
    scratch += [pltpu.VMEM_SHARED((NPAD, CW), jnp.float32),
                pltpu.VMEM((CH, CW), jnp.float32)]
    args += [zcnt, ones]

  kern = pl.kernel(body, out_type=tuple(out_type), mesh=_mesh(),
                   compiler_params=_SC_PARAMS, scratch_types=scratch)
  return kern(*args)


def _tc_layer(x, s2, c2, ws, wn, pp):
  Din = x.shape[1]

  def body(xr, sr, cr, wsr, wnr, ppr, hr, gr):
    sv = sr[...]
    cv = cr[...]
    cnt = cv[0][:, 0:1] + cv[1][:, 0:1]
    hn = (sv[0] + sv[1]) / jnp.maximum(cnt, 1.0)
    r = jnp.dot(xr[...], wsr[...], preferred_element_type=jnp.float32)
    r = r + jnp.dot(hn, wnr[...], preferred_element_type=jnp.float32)
    r = r + ppr[0:1, :]
    r = jnp.where(r > 0, r, ppr[1:2, :] * r)
    r = ppr[2:3, :] * r + ppr[3:4, :]
    hr[...] = r

    @pl.when(pl.program_id(0) == 0)
    def _():
      gr[...] = jnp.zeros_like(gr)

    gr[...] += jnp.sum(r, axis=0, keepdims=True) * (1.0 / N)

  return pl.pallas_call(
      body,
      grid=(N // BLK,),
      in_specs=[
          pl.BlockSpec((BLK, Din), lambda i: (i, 0)),
          pl.BlockSpec((NC, BLK, Din), lambda i: (0, i, 0)),
          pl.BlockSpec((NC, BLK, CW), lambda i: (0, i, 0)),
          pl.BlockSpec((Din, H), lambda i: (0, 0)),
          pl.BlockSpec((Din, H), lambda i: (0, 0)),
          pl.BlockSpec((8, H), lambda i: (0, 0)),
      ],
      out_specs=[
          pl.BlockSpec((BLK, H), lambda i: (i, 0)),
          pl.BlockSpec((1, H), lambda i: (0, 0)),
      ],
      out_shape=[
          jax.ShapeDtypeStruct((N, H), jnp.float32),
          jax.ShapeDtypeStruct((1, H), jnp.float32),
      ],
  )(x, s2, c2, ws, wn, pp)


def kernel(feat_h, feat_p, feat_hp, eidx_h, eidx_p, eidx_hp,
           emb_h, emb_p, emb_hp, Ws1, Wn1, b1, a1, g1, be1,
           Ws, Wn, b, a, g, be):
  f = jnp.stack([feat_h, feat_p, feat_hp]).astype(jnp.int32)
  f = jnp.pad(f, ((0, 0), (0, N_PADL - N)))
  f_r = f.reshape(3, NW, NLC, LC)

  srcs, dsts = [], []
  for eidx in (eidx_h, eidx_p, eidx_hp):
    ei = eidx.astype(jnp.int32)
    srcp = jnp.pad(ei[0], (0, E_PAD - E))
    dstp = jnp.pad(ei[1], (0, E_PAD - E), constant_values=N)
    srcs.append(srcp.reshape(NW, NCHUNK, CH))
    dsts.append(dstp.reshape(NW, NCHUNK, CH))

  zcnt = jnp.zeros((RPT, CW), jnp.float32)
  ones = jnp.ones((CH, CW), jnp.float32)

  x0 = _emb_lookup(emb_h, emb_p, emb_hp, f_r)

  def pack_params(bb, aa, gg, bee):
    return jnp.concatenate(
        [jnp.stack([bb, aa, gg, bee]), jnp.zeros((4, H), jnp.float32)], axis=0)

  gsums = []
  for t in range(3):
    xt = x0[t]
    s2, c2 = _seg_sum(EMB, True, xt, srcs[t], dsts[t], zcnt, ones)
    h, gs = _tc_layer(xt, s2, c2, Ws1[t], Wn1[t],
                      pack_params(b1[t], a1[t], g1[t], be1[t]))
    t_gs = [gs]
    for l in range(3):
      (s2,) = _seg_sum(H, False, h, srcs[t], dsts[t])
      h, gs = _tc_layer(h, s2, c2, Ws[l, t], Wn[l, t],
                        pack_params(b[l, t], a[l, t], g[l, t], be[l, t]))
      t_gs.append(gs)
    gsums.append(t_gs)

  g_vec = jnp.concatenate(
      [gsums[t][l].reshape(H) for t in range(3) for l in range(4)])
  return g_vec.reshape(1, 12 * H)
